# baseline (speedup 1.0000x reference)
import jax
import jax.numpy as jnp
from jax import lax
from jax.experimental import pallas as pl
from jax.experimental.pallas import tpu as pltpu
from jax.experimental.pallas import tpu_sc as plsc

_NC = 2
_NS = 16
_NW = _NC * _NS
_C = 8
_K = 4


def _pe_gather(pe_hbm, idx_hbm, out_hbm, idx_v,
               rows0, rows1, rows2, rows3,
               g0, g1, g2, g3, o0, o1, o2, o3):
    n = idx_hbm.shape[0]
    per_w = n // _NW
    nchunk = per_w // _C
    wid = lax.axis_index("s") * _NC + lax.axis_index("c")
    base = wid * per_w
    pltpu.sync_copy(idx_hbm.at[pl.ds(base, per_w)], idx_v)

    rows = (rows0, rows1, rows2, rows3)
    gsem = (g0, g1, g2, g3)
    osem = (o0, o1, o2, o3)

    def gather(c, b):
        pltpu.async_copy(pe_hbm.at[idx_v.at[pl.ds(c * _C, _C)]], rows[b], gsem[b])

    def wait_gather(b):
        pltpu.make_async_copy(
            pe_hbm.at[idx_v.at[pl.ds(0, _C)]], rows[b], gsem[b]).wait()

    def put(i, b):
        pltpu.async_copy(rows[b], out_hbm.at[pl.ds(base + i * _C, _C)], osem[b])

    def wait_put(b):
        pltpu.make_async_copy(
            rows[b], out_hbm.at[pl.ds(base, _C)], osem[b]).wait()

    gather(0, 0)
    gather(1, 1)

    wait_gather(0); put(0, 0); gather(2, 2)
    wait_gather(1); put(1, 1); gather(3, 3)
    wait_gather(2); put(2, 2); wait_put(0); gather(4, 0)
    wait_gather(3); put(3, 3); wait_put(1); gather(5, 1)

    def body(q, carry):
        i0 = 4 * q
        for b in range(_K):
            i = i0 + b
            b2 = (b + 2) % _K
            wait_gather(b)
            put(i, b)
            wait_put(b2)
            gather(i + 2, b2)
        return carry

    lax.fori_loop(1, nchunk // _K - 1, body, 0)

    i0 = nchunk - 4
    wait_gather(0); put(i0 + 0, 0); wait_put(2); gather(i0 + 2, 2)
    wait_gather(1); put(i0 + 1, 1); wait_put(3); gather(i0 + 3, 3)
    wait_gather(2); put(i0 + 2, 2)
    wait_gather(3); put(i0 + 3, 3)

    wait_put(0); wait_put(1); wait_put(2); wait_put(3)


def kernel(pe, frame_idx):
    B, S = frame_idx.shape
    V, D = pe.shape
    flat_idx = frame_idx.reshape(B * S)
    per_w = (B * S) // _NW
    run = pl.kernel(
        _pe_gather,
        out_type=jax.ShapeDtypeStruct((B * S, D), pe.dtype),
        mesh=plsc.VectorSubcoreMesh(core_axis_name="c", subcore_axis_name="s"),
        scratch_types=[
            pltpu.VMEM((per_w,), jnp.int32),
            pltpu.VMEM((_C, D), jnp.float32),
            pltpu.VMEM((_C, D), jnp.float32),
            pltpu.VMEM((_C, D), jnp.float32),
            pltpu.VMEM((_C, D), jnp.float32),
            pltpu.SemaphoreType.DMA,
            pltpu.SemaphoreType.DMA,
            pltpu.SemaphoreType.DMA,
            pltpu.SemaphoreType.DMA,
            pltpu.SemaphoreType.DMA,
            pltpu.SemaphoreType.DMA,
            pltpu.SemaphoreType.DMA,
            pltpu.SemaphoreType.DMA,
        ],
    )
    out = run(pe, flat_idx)
    return out.reshape(B, S, D)
